# baseline (speedup 1.0000x reference)
import functools
import math

import jax
import jax.numpy as jnp
from jax import lax
from jax.experimental import pallas as pl
from jax.experimental.pallas import tpu as pltpu
from jax.experimental.pallas import tpu_sc as plsc

_D = 64
_LANES = 16
_NC = 2
_NS = 16
_NW = _NC * _NS
_B = 8
_CHUNK = 128
_SCALE = math.sqrt(_D)


@functools.lru_cache(maxsize=None)
def _build(n_j: int, n_b: int):
    npt = _B * n_j
    assert n_j <= _CHUNK
    n_tasks = n_b // _B
    assert n_tasks % _NW == 0
    tpw = n_tasks // _NW

    mesh = plsc.VectorSubcoreMesh(
        core_axis_name="c", subcore_axis_name="s",
        num_cores=_NC, num_subcores=_NS)

    @functools.partial(
        pl.kernel,
        out_type=jax.ShapeDtypeStruct((n_b, n_j, _D), jnp.float32),
        mesh=mesh,
        compiler_params=pltpu.CompilerParams(
            use_tc_tiling_on_sc=False, needs_layout_passes=False),
        scratch_types=[
            pltpu.VMEM((2, _B, n_j), jnp.int32),
            pltpu.VMEM((2, _B, n_j, _D), jnp.float32),
            pltpu.SemaphoreType.DMA((2,)),
            pltpu.SemaphoreType.DMA((2,)),
            pltpu.SemaphoreType.DMA((2,)),
        ],
    )
    def sc_embed(idx_hbm, table_hbm, out_hbm, idx_v, rows_v,
                 isem, gsem, wsem):
        wid = lax.axis_index("s") * _NC + lax.axis_index("c")
        t0 = wid * tpw

        def idx_src(t):
            return idx_hbm.at[pl.ds((t0 + t) * _B, _B)]

        def out_dst(t):
            return out_hbm.at[pl.ds((t0 + t) * _B, _B)]

        def fire_idx(t, p):
            pltpu.async_copy(idx_src(t), idx_v.at[p], isem.at[p])

        def wait_idx(t, p):
            pltpu.make_async_copy(idx_src(t), idx_v.at[p],
                                  isem.at[p]).wait()

        def fire_gather(t, p):
            for r in range(_B):
                pltpu.async_copy(
                    table_hbm.at[idx_v.at[p, r]],
                    rows_v.at[p, r], gsem.at[p])

        def wait_gather(t, p):
            for r in range(_B):
                pltpu.make_async_copy(
                    table_hbm.at[idx_v.at[p, r]],
                    rows_v.at[p, r], gsem.at[p]).wait()

        def fire_write(t, p):
            pltpu.async_copy(rows_v.at[p], out_dst(t), wsem.at[p])

        def wait_write(t, p):
            pltpu.make_async_copy(rows_v.at[p], out_dst(t),
                                  wsem.at[p]).wait()

        fire_idx(0, 0)
        fire_idx(1, 1)
        wait_idx(0, 0)
        fire_gather(0, 0)

        def pair_body(t2, carry):
            for p in range(2):
                t = t2 * 2 + p
                q = 1 - p
                @pl.when(t + 1 < tpw)
                def _():
                    wait_idx(t + 1, q)

                    @pl.when(t >= 1)
                    def _():
                        wait_write(t - 1, q)

                    fire_gather(t + 1, q)

                wait_gather(t, p)

                @pl.when(t + 2 < tpw)
                def _():
                    fire_idx(t + 2, p)

                def sc_body(m, carry2):
                    r = m // n_j
                    jj = m % n_j
                    for k in range(_D // _LANES):
                        rows_v[p, r, jj, pl.ds(k * _LANES, _LANES)] = (
                            rows_v[p, r, jj, pl.ds(k * _LANES, _LANES)]
                            * _SCALE)
                    return carry2

                lax.fori_loop(0, npt, sc_body, 0, unroll=4)
                fire_write(t, p)
            return carry

        lax.fori_loop(0, tpw // 2, pair_body, 0)
        wait_write(tpw - 2, 0)
        wait_write(tpw - 1, 1)

    return sc_embed


def kernel(x, table):
    n_b, n_j = x.shape
    return _build(n_j, n_b)(x, table)


# speedup vs baseline: 1.0654x; 1.0654x over previous
import functools
import math

import jax
import jax.numpy as jnp
from jax import lax
from jax.experimental import pallas as pl
from jax.experimental.pallas import tpu as pltpu
from jax.experimental.pallas import tpu_sc as plsc

_D = 64
_LANES = 16
_NC = 2
_NS = 16
_NW = _NC * _NS
_B = 8
_CHUNK = 128
_SCALE = math.sqrt(_D)


@functools.lru_cache(maxsize=None)
def _build(n_j: int, n_b: int):
    npt = _B * n_j
    n_tasks = n_b // _B
    assert n_tasks % _NW == 0
    tpw = n_tasks // _NW
    chunks = []
    off = 0
    while off < npt:
        c = min(_CHUNK, npt - off)
        chunks.append((off, c))
        off += c

    mesh = plsc.VectorSubcoreMesh(
        core_axis_name="c", subcore_axis_name="s",
        num_cores=_NC, num_subcores=_NS)

    @functools.partial(
        pl.kernel,
        out_type=jax.ShapeDtypeStruct((n_tasks, npt, _D), jnp.float32),
        mesh=mesh,
        compiler_params=pltpu.CompilerParams(
            use_tc_tiling_on_sc=False, needs_layout_passes=False),
        scratch_types=[
            pltpu.VMEM((2, npt), jnp.int32),
            pltpu.VMEM((2, npt, _D), jnp.float32),
            pltpu.SemaphoreType.DMA((2,)),
            pltpu.SemaphoreType.DMA((2,)),
            pltpu.SemaphoreType.DMA((2,)),
        ],
    )
    def sc_embed(idx_hbm, table_hbm, out_hbm, idx_v, rows_v,
                 isem, gsem, wsem):
        wid = lax.axis_index("s") * _NC + lax.axis_index("c")
        t0 = wid * tpw

        def fire_idx(t, p):
            pltpu.async_copy(idx_hbm.at[t0 + t], idx_v.at[p], isem.at[p])

        def wait_idx(t, p):
            pltpu.make_async_copy(idx_hbm.at[t0 + t], idx_v.at[p],
                                  isem.at[p]).wait()

        def fire_gather(t, p):
            for off, c in chunks:
                pltpu.async_copy(
                    table_hbm.at[idx_v.at[p, pl.ds(off, c)]],
                    rows_v.at[p, pl.ds(off, c)], gsem.at[p])

        def wait_gather(t, p):
            for off, c in chunks:
                pltpu.make_async_copy(
                    table_hbm.at[idx_v.at[p, pl.ds(off, c)]],
                    rows_v.at[p, pl.ds(off, c)], gsem.at[p]).wait()

        def fire_write(t, p):
            pltpu.async_copy(rows_v.at[p], out_hbm.at[t0 + t], wsem.at[p])

        def wait_write(t, p):
            pltpu.make_async_copy(rows_v.at[p], out_hbm.at[t0 + t],
                                  wsem.at[p]).wait()

        fire_idx(0, 0)
        fire_idx(1, 1)
        wait_idx(0, 0)
        fire_gather(0, 0)

        def pair_body(t2, carry):
            for p in range(2):
                t = t2 * 2 + p
                q = 1 - p
                @pl.when(t + 1 < tpw)
                def _():
                    wait_idx(t + 1, q)

                    @pl.when(t >= 1)
                    def _():
                        wait_write(t - 1, q)

                    fire_gather(t + 1, q)

                wait_gather(t, p)

                @pl.when(t + 2 < tpw)
                def _():
                    fire_idx(t + 2, p)

                def sc_body(m, carry2):
                    r = m // (_D // _LANES)
                    cc = (m % (_D // _LANES)) * _LANES
                    rows_v[p, r, pl.ds(cc, _LANES)] = (
                        rows_v[p, r, pl.ds(cc, _LANES)] * _SCALE)
                    return carry2

                lax.fori_loop(0, (npt * _D) // _LANES, sc_body, 0,
                              unroll=8)
                fire_write(t, p)
            return carry

        lax.fori_loop(0, tpw // 2, pair_body, 0)
        wait_write(tpw - 2, 0)
        wait_write(tpw - 1, 1)

    return sc_embed


def kernel(x, table):
    n_b, n_j = x.shape
    idx2 = x.reshape(n_b // _B, _B * n_j)
    out = _build(n_j, n_b)(idx2, table)
    return out.reshape(n_b, n_j, _D)
